# baseline (speedup 1.0000x reference)
import functools

import jax
import jax.numpy as jnp
from jax import lax
from jax.experimental import pallas as pl
from jax.experimental.pallas import tpu as pltpu
from jax.experimental.pallas import tpu_sc as plsc

D = 32
NT = 4
CHUNK = 128

_info = plsc.get_sparse_core_info()
_NC, _NS = _info.num_cores, _info.num_subcores
NW = _NC * _NS


def _make_kernel(batch):
    b_per_w = batch // NW
    nchunk = b_per_w // CHUNK

    mesh = plsc.VectorSubcoreMesh(core_axis_name="c", subcore_axis_name="s")

    @functools.partial(
        pl.kernel,
        mesh=mesh,
        out_type=jax.ShapeDtypeStruct((batch, NT * D), jnp.float32),
        scratch_types=[
            pltpu.VMEM((NT, nchunk, CHUNK), jnp.int32),
            pltpu.VMEM((NT * nchunk, CHUNK, D), jnp.float32),
            pltpu.SemaphoreType.DMA,
        ],
        compiler_params=pltpu.CompilerParams(use_tc_tiling_on_sc=False),
    )
    def emb_kernel(g_hbm, a_hbm, o_hbm, z_hbm, Wg, Wa, Wo, Wz, out_hbm,
                   idx, gbuf, sem):
        wid = lax.axis_index("s") * _NC + lax.axis_index("c")
        base = wid * nchunk
        pltpu.sync_copy(g_hbm.at[pl.ds(base, nchunk)], idx.at[0])
        pltpu.sync_copy(a_hbm.at[pl.ds(base, nchunk)], idx.at[1])
        pltpu.sync_copy(o_hbm.at[pl.ds(base, nchunk)], idx.at[2])
        pltpu.sync_copy(z_hbm.at[pl.ds(base, nchunk)], idx.at[3])
        copies = []
        for t, W in enumerate((Wg, Wa, Wo, Wz)):
            for j in range(nchunk):
                dst = gbuf.at[t * nchunk + j]
                copies.append(pltpu.async_copy(W.at[idx.at[t].at[j]], dst, sem))
        for c in copies:
            c.wait()
        row0 = wid * b_per_w
        for t in range(NT):
            for j in range(nchunk):
                dst = out_hbm.at[pl.ds(row0 + j * CHUNK, CHUNK), pl.ds(t * D, D)]
                pltpu.sync_copy(gbuf.at[t * nchunk + j], dst)

    return emb_kernel


def kernel(gender_idx, age_idx, occupation_idx, area_idx,
           W_gender, W_age, W_occupation, W_area):
    batch = gender_idx.shape[0]
    shape2d = (batch // CHUNK, CHUNK)
    g = gender_idx.astype(jnp.int32).reshape(shape2d)
    a = age_idx.astype(jnp.int32).reshape(shape2d)
    o = occupation_idx.astype(jnp.int32).reshape(shape2d)
    z = area_idx.astype(jnp.int32).reshape(shape2d)
    return _make_kernel(batch)(g, a, o, z, W_gender, W_age, W_occupation, W_area)


# speedup vs baseline: 2.1912x; 2.1912x over previous
import functools

import jax
import jax.numpy as jnp
from jax import lax
from jax.experimental import pallas as pl
from jax.experimental.pallas import tpu as pltpu
from jax.experimental.pallas import tpu_sc as plsc

D = 32
NT = 4
CHUNK = 128
L = 16

_info = plsc.get_sparse_core_info()
_NC, _NS = _info.num_cores, _info.num_subcores
NW = _NC * _NS


def _make_kernel(batch):
    b_per_w = batch // NW
    nchunk = b_per_w // CHUNK

    mesh = plsc.VectorSubcoreMesh(core_axis_name="c", subcore_axis_name="s")

    @functools.partial(
        pl.kernel,
        mesh=mesh,
        out_type=jax.ShapeDtypeStruct((batch, NT * D), jnp.float32),
        scratch_types=[
            pltpu.VMEM((NT, nchunk, CHUNK), jnp.int32),
            pltpu.VMEM((NT * nchunk, CHUNK, D), jnp.float32),
            pltpu.VMEM((30, D), jnp.float32),
            pltpu.SemaphoreType.DMA,
        ],
        compiler_params=pltpu.CompilerParams(
            use_tc_tiling_on_sc=False, needs_layout_passes=False),
    )
    def emb_kernel(g_hbm, a_hbm, o_hbm, z_hbm, Wg, Wa, Wo, Wz, out_hbm,
                   idx, gbuf, smalls, sem):
        wid = lax.axis_index("s") * _NC + lax.axis_index("c")
        base = wid * nchunk
        pltpu.sync_copy(Wg, smalls.at[pl.ds(0, 2)])
        pltpu.sync_copy(Wa, smalls.at[pl.ds(2, 7)])
        pltpu.sync_copy(Wo, smalls.at[pl.ds(9, 21)])
        pltpu.sync_copy(g_hbm.at[pl.ds(base, nchunk)], idx.at[0])
        pltpu.sync_copy(a_hbm.at[pl.ds(base, nchunk)], idx.at[1])
        pltpu.sync_copy(o_hbm.at[pl.ds(base, nchunk)], idx.at[2])
        pltpu.sync_copy(z_hbm.at[pl.ds(base, nchunk)], idx.at[3])
        copies = []
        for j in range(nchunk):
            dst = gbuf.at[3 * nchunk + j]
            copies.append(pltpu.async_copy(Wz.at[idx.at[3].at[j]], dst, sem))
        lane = lax.iota(jnp.int32, L)
        for t, roff in ((0, 0), (1, 2), (2, 9)):
            def body(k, _, t=t, roff=roff):
                j = k // (CHUNK // L)
                g = k % (CHUNK // L)
                rows = idx[t, j, pl.ds(g * L, L)] + roff
                slot = t * nchunk + j
                erow = g * L + lane
                for c in range(D):
                    vals = plsc.load_gather(
                        smalls, [rows, jnp.full((L,), c, jnp.int32)])
                    plsc.store_scatter(
                        gbuf,
                        [jnp.full((L,), slot, jnp.int32), erow,
                         jnp.full((L,), c, jnp.int32)],
                        vals)
                return ()
            lax.fori_loop(0, nchunk * (CHUNK // L), body, ())
        for c in copies:
            c.wait()
        row0 = wid * b_per_w
        for t in range(NT):
            for j in range(nchunk):
                dst = out_hbm.at[pl.ds(row0 + j * CHUNK, CHUNK), pl.ds(t * D, D)]
                pltpu.sync_copy(gbuf.at[t * nchunk + j], dst)

    return emb_kernel


def kernel(gender_idx, age_idx, occupation_idx, area_idx,
           W_gender, W_age, W_occupation, W_area):
    batch = gender_idx.shape[0]
    shape2d = (batch // CHUNK, CHUNK)
    g = gender_idx.astype(jnp.int32).reshape(shape2d)
    a = age_idx.astype(jnp.int32).reshape(shape2d)
    o = occupation_idx.astype(jnp.int32).reshape(shape2d)
    z = area_idx.astype(jnp.int32).reshape(shape2d)
    return _make_kernel(batch)(g, a, o, z, W_gender, W_age, W_occupation, W_area)
